# baseline (speedup 1.0000x reference)
import functools

import jax
import jax.numpy as jnp
from jax import lax
from jax.experimental import pallas as pl
from jax.experimental.pallas import tpu as pltpu
from jax.experimental.pallas import tpu_sc as plsc

_LANES = 16
_CHUNK = 128


@functools.partial(jax.jit, static_argnames=("b_per_w", "num_cores", "num_sub"))
def _sc_time_embedding(timestep, scale, pe_matrix, *, b_per_w, num_cores, num_sub):
    B = timestep.shape[0]
    D = pe_matrix.shape[1]
    n_chunks = b_per_w // _CHUNK
    mesh = plsc.VectorSubcoreMesh(core_axis_name="c", subcore_axis_name="s")

    V = pe_matrix.shape[0]

    @functools.partial(
        pl.kernel,
        mesh=mesh,
        out_type=jax.ShapeDtypeStruct((B, D), jnp.float32),
        scratch_types=[
            pltpu.VMEM((b_per_w,), jnp.float32),
            pltpu.VMEM((_LANES,), jnp.float32),
            pltpu.VMEM((n_chunks, _CHUNK), jnp.int32),
            pltpu.VMEM((b_per_w, D), jnp.float32),
            pltpu.VMEM_SHARED((V, D), jnp.float32),
            pltpu.SemaphoreType.DMA,
            pltpu.SemaphoreType.DMA,
            pltpu.SemaphoreType.DMA,
        ],
    )
    def body(ts_hbm, scale_hbm, table_hbm, out_hbm, ts_v, scale_v, idx_v, rows_v,
             table_sh, gsem, hsem, wsem):
        sid = lax.axis_index("s")
        wid = sid * num_cores + lax.axis_index("c")
        base = wid * b_per_w
        @pl.when(sid == 0)
        def _():
            pltpu.sync_copy(table_hbm, table_sh)
        pltpu.sync_copy(scale_hbm, scale_v)
        pltpu.sync_copy(ts_hbm.at[pl.ds(base, b_per_w)], ts_v)
        scale = scale_v[...]

        def compute_idx(i, carry):
            t = ts_v[pl.ds(pl.multiple_of(i * _LANES, _LANES), _LANES)]
            iv = (t * scale).astype(jnp.int32)
            c = i // (_CHUNK // _LANES)
            j = lax.rem(i, _CHUNK // _LANES)
            idx_v[c, pl.ds(pl.multiple_of(j * _LANES, _LANES), _LANES)] = iv
            return carry

        lax.fori_loop(0, b_per_w // _LANES, compute_idx, 0)
        plsc.subcore_barrier()
        gathers = [
            pltpu.async_copy(
                (table_hbm if c % 2 == 0 else table_sh).at[idx_v.at[c]],
                rows_v.at[pl.ds(c * _CHUNK, _CHUNK)],
                hsem if c % 2 == 0 else gsem,
            )
            for c in range(n_chunks)
        ]
        writes = []
        for c in range(n_chunks):
            gathers[c].wait()
            writes.append(
                pltpu.async_copy(
                    rows_v.at[pl.ds(c * _CHUNK, _CHUNK)],
                    out_hbm.at[pl.ds(base + c * _CHUNK, _CHUNK)],
                    wsem,
                )
            )
        for w in writes:
            w.wait()

    return body(timestep, scale, pe_matrix)


def kernel(timestep, T, pe_matrix):
    info = plsc.get_sparse_core_info()
    num_workers = info.num_cores * info.num_subcores
    B = timestep.shape[0]
    b_per_w = B // num_workers
    scale = jnp.broadcast_to(jnp.asarray(T, jnp.float32), (_LANES,))
    return _sc_time_embedding(
        timestep, scale, pe_matrix, b_per_w=b_per_w,
        num_cores=info.num_cores, num_sub=info.num_subcores,
    )


# speedup vs baseline: 1.1500x; 1.1500x over previous
import functools

import jax
import jax.numpy as jnp
from jax import lax
from jax.experimental import pallas as pl
from jax.experimental.pallas import tpu as pltpu
from jax.experimental.pallas import tpu_sc as plsc

_LANES = 16
_CHUNK = 128


@functools.partial(jax.jit, static_argnames=("b_per_w", "num_cores", "num_sub"))
def _sc_time_embedding(timestep, scale, pe_matrix, *, b_per_w, num_cores, num_sub):
    B = timestep.shape[0]
    D = pe_matrix.shape[1]
    n_chunks = b_per_w // _CHUNK
    mesh = plsc.VectorSubcoreMesh(core_axis_name="c", subcore_axis_name="s")

    V = pe_matrix.shape[0]

    @functools.partial(
        pl.kernel,
        mesh=mesh,
        out_type=jax.ShapeDtypeStruct((B, D), jnp.float32),
        scratch_types=[
            pltpu.VMEM((b_per_w,), jnp.float32),
            pltpu.VMEM((_LANES,), jnp.float32),
            pltpu.VMEM((n_chunks, _CHUNK), jnp.int32),
            pltpu.VMEM((b_per_w, D), jnp.float32),
            pltpu.VMEM_SHARED((V, D), jnp.float32),
            pltpu.SemaphoreType.DMA,
            pltpu.SemaphoreType.DMA,
        ],
    )
    def body(ts_hbm, scale_hbm, table_hbm, out_hbm, ts_v, scale_v, idx_v, rows_v,
             table_sh, gsem, wsem):
        sid = lax.axis_index("s")
        wid = sid * num_cores + lax.axis_index("c")
        base = wid * b_per_w
        @pl.when(sid == 0)
        def _():
            pltpu.sync_copy(table_hbm, table_sh.at[:, pl.ds(0, D)])
        pltpu.sync_copy(scale_hbm, scale_v)
        pltpu.sync_copy(ts_hbm.at[pl.ds(base, b_per_w)], ts_v)
        scale = scale_v[...]

        def compute_idx(i, carry):
            t = ts_v[pl.ds(pl.multiple_of(i * _LANES, _LANES), _LANES)]
            iv = (t * scale).astype(jnp.int32)
            c = i // (_CHUNK // _LANES)
            j = lax.rem(i, _CHUNK // _LANES)
            idx_v[c, pl.ds(pl.multiple_of(j * _LANES, _LANES), _LANES)] = iv
            return carry

        lax.fori_loop(0, b_per_w // _LANES, compute_idx, 0)
        plsc.subcore_barrier()
        gathers = [
            pltpu.async_copy(
                table_sh.at[idx_v.at[c]],
                rows_v.at[pl.ds(c * _CHUNK, _CHUNK)],
                gsem,
            )
            for c in range(n_chunks)
        ]
        writes = []
        for c in range(n_chunks):
            gathers[c].wait()
            writes.append(
                pltpu.async_copy(
                    rows_v.at[pl.ds(c * _CHUNK, _CHUNK), pl.ds(0, D)],
                    out_hbm.at[pl.ds(base + c * _CHUNK, _CHUNK)],
                    wsem,
                )
            )
        for w in writes:
            w.wait()

    return body(timestep, scale, pe_matrix)


def kernel(timestep, T, pe_matrix):
    info = plsc.get_sparse_core_info()
    num_workers = info.num_cores * info.num_subcores
    B = timestep.shape[0]
    b_per_w = B // num_workers
    scale = jnp.broadcast_to(jnp.asarray(T, jnp.float32), (_LANES,))
    return _sc_time_embedding(
        timestep, scale, pe_matrix, b_per_w=b_per_w,
        num_cores=info.num_cores, num_sub=info.num_subcores,
    )
